# Initial kernel scaffold; baseline (speedup 1.0000x reference)
import functools

import jax
import jax.numpy as jnp
from jax import lax
from jax.experimental import pallas as pl
from jax.experimental.pallas import tpu as pltpu
from jax.experimental.pallas import tpu_sc as plsc

N = 10000
E = 320000
IN_CH = 128
OUT_CH = 128

FEAT = 144
NPAD = 10112
ROWS_PER_SUB = 632
NUM_CORES = 2
NUM_SUBCORES = 16
NW = NUM_CORES * NUM_SUBCORES
CHUNK = 128
CPW = 80
EPAD = NW * CPW * CHUNK
ZROWS = 79


def _sc_segment_sum(xaug, src2d, dst2d):
  mesh = plsc.VectorSubcoreMesh(
      core_axis_name="c", subcore_axis_name="s",
      num_cores=NUM_CORES, num_subcores=NUM_SUBCORES)

  @functools.partial(
      pl.kernel,
      out_type=jax.ShapeDtypeStruct((NUM_CORES, NPAD, FEAT), jnp.float32),
      mesh=mesh,
      scratch_types=[
          pltpu.VMEM((CPW, CHUNK), jnp.int32),
          pltpu.VMEM((CPW, CHUNK), jnp.int32),
          pltpu.VMEM((CHUNK, FEAT), jnp.float32),
          pltpu.VMEM((ZROWS, FEAT), jnp.float32),
          pltpu.VMEM_SHARED((NPAD, FEAT), jnp.float32),
          pltpu.SemaphoreType.DMA,
      ],
  )
  def k(xaug_hbm, src_hbm, dst_hbm, out_hbm, src_v, dst_v, gb, zb, acc, sem):
    cid = lax.axis_index("c")
    sid = lax.axis_index("s")
    wid = cid * NUM_SUBCORES + sid
    base = sid * ROWS_PER_SUB

    zeros16 = jnp.zeros((16,), jnp.float32)

    def zrow(r, carry):
      for j in range(FEAT // 16):
        zb[r, pl.ds(j * 16, 16)] = zeros16
      return carry

    lax.fori_loop(0, ZROWS, zrow, 0)
    for t in range(ROWS_PER_SUB // ZROWS):
      pltpu.sync_copy(zb, acc.at[pl.ds(base + t * ZROWS, ZROWS)])

    pltpu.sync_copy(src_hbm.at[pl.ds(wid * CPW, CPW)], src_v)
    pltpu.sync_copy(dst_hbm.at[pl.ds(wid * CPW, CPW)], dst_v)

    plsc.subcore_barrier()

    def body(c, carry):
      pltpu.async_copy(xaug_hbm.at[src_v.at[c]], gb, sem).wait()
      pltpu.sync_copy(gb, acc.at[dst_v.at[c]], add=True)
      return carry

    lax.fori_loop(0, CPW, body, 0)

    plsc.subcore_barrier()

    pltpu.sync_copy(acc.at[pl.ds(base, ROWS_PER_SUB)],
                    out_hbm.at[cid, pl.ds(base, ROWS_PER_SUB)])

  return k(xaug, src2d, dst2d)


def _tc_combine_body(p_ref, x_ref, wl_ref, wr_ref, b_ref, o_ref):
  s = p_ref[0, :, :IN_CH] + p_ref[1, :, :IN_CH]
  cnt = p_ref[0, :, IN_CH:IN_CH + 1] + p_ref[1, :, IN_CH:IN_CH + 1]
  mean = s / jnp.maximum(cnt, 1.0)
  xb = x_ref[:, :IN_CH]
  dims = (((1,), (1,)), ((), ()))
  o_ref[...] = (
      lax.dot_general(mean, wl_ref[...], dims,
                      preferred_element_type=jnp.float32)
      + b_ref[...]
      + lax.dot_general(xb, wr_ref[...], dims,
                        preferred_element_type=jnp.float32))


def _tc_combine(part, xaug, w_l, w_r, b_l):
  blk = 1264
  grid = NPAD // blk
  return pl.pallas_call(
      _tc_combine_body,
      grid=(grid,),
      in_specs=[
          pl.BlockSpec((NUM_CORES, blk, FEAT), lambda i: (0, i, 0)),
          pl.BlockSpec((blk, FEAT), lambda i: (i, 0)),
          pl.BlockSpec((OUT_CH, IN_CH), lambda i: (0, 0)),
          pl.BlockSpec((OUT_CH, IN_CH), lambda i: (0, 0)),
          pl.BlockSpec((1, OUT_CH), lambda i: (0, 0)),
      ],
      out_specs=pl.BlockSpec((blk, OUT_CH), lambda i: (i, 0)),
      out_shape=jax.ShapeDtypeStruct((NPAD, OUT_CH), jnp.float32),
  )(part, xaug, w_l, w_r, b_l)


def kernel(x, edge_index, W_l, b_l, W_r):
  src = edge_index[0]
  dst = edge_index[1]
  pad = EPAD - E
  src_p = jnp.concatenate([src, jnp.zeros((pad,), jnp.int32)])
  dst_pad = N + (jnp.arange(pad, dtype=jnp.int32) % (NPAD - N))
  dst_p = jnp.concatenate([dst, dst_pad])
  src2d = src_p.reshape(NW * CPW, CHUNK)
  dst2d = dst_p.reshape(NW * CPW, CHUNK)

  xaug = jnp.zeros((NPAD, FEAT), jnp.float32)
  xaug = xaug.at[:N, :IN_CH].set(x)
  xaug = xaug.at[:N, IN_CH].set(1.0)

  part = _sc_segment_sum(xaug, src2d, dst2d)
  out = _tc_combine(part, xaug, W_l, W_r, b_l.reshape(1, OUT_CH))
  return out[:N]


# speedup vs baseline: 3.5534x; 3.5534x over previous
import functools

import jax
import jax.numpy as jnp
from jax import lax
from jax.experimental import pallas as pl
from jax.experimental.pallas import tpu as pltpu
from jax.experimental.pallas import tpu_sc as plsc

N = 10000
E = 320000
IN_CH = 128
OUT_CH = 128

FEAT = 144
NPAD = 10112
ROWS_PER_SUB = 632
NUM_CORES = 2
NUM_SUBCORES = 16
NW = NUM_CORES * NUM_SUBCORES
CHUNK = 128
CPW = 80
EPAD = NW * CPW * CHUNK
STAGE = 8
NSTAGE = CPW // STAGE


def _sc_segment_sum(xaug, src2d, dst2d):
  mesh = plsc.VectorSubcoreMesh(
      core_axis_name="c", subcore_axis_name="s",
      num_cores=NUM_CORES, num_subcores=NUM_SUBCORES)

  @functools.partial(
      pl.kernel,
      out_type=jax.ShapeDtypeStruct((NUM_CORES, NPAD, FEAT), jnp.float32),
      mesh=mesh,
      compiler_params=pltpu.CompilerParams(use_tc_tiling_on_sc=False),
      scratch_types=[
          pltpu.VMEM((STAGE, CHUNK), jnp.int32),
          pltpu.VMEM((STAGE, CHUNK), jnp.int32),
          pltpu.VMEM((CHUNK, FEAT), jnp.float32),
          pltpu.VMEM((CHUNK, FEAT), jnp.float32),
          pltpu.VMEM_SHARED((NPAD, FEAT), jnp.float32),
          pltpu.SemaphoreType.DMA,
          pltpu.SemaphoreType.DMA,
      ],
  )
  def k(xaug_hbm, src_hbm, dst_hbm, out_hbm, src_v, dst_v, gb0, gb1, acc,
        sem0, sem1):
    cid = lax.axis_index("c")
    sid = lax.axis_index("s")
    wid = cid * NUM_SUBCORES + sid
    base = sid * ROWS_PER_SUB

    zeros16 = jnp.zeros((16,), jnp.float32)

    def zrow(r, carry):
      for j in range(FEAT // 16):
        gb0[r, pl.ds(j * 16, 16)] = zeros16
      return carry

    lax.fori_loop(0, CHUNK, zrow, 0)
    for t in range(4):
      pltpu.sync_copy(gb0, acc.at[pl.ds(base + t * CHUNK, CHUNK)])
    pltpu.sync_copy(gb0.at[pl.ds(0, ROWS_PER_SUB - 4 * CHUNK)],
                    acc.at[pl.ds(base + 4 * CHUNK, ROWS_PER_SUB - 4 * CHUNK)])

    plsc.subcore_barrier()

    gbs = (gb0, gb1)
    sems = (sem0, sem1)

    def stage_body(s, carry):
      row0 = wid * CPW + s * STAGE
      pltpu.sync_copy(src_hbm.at[pl.ds(row0, STAGE)], src_v)
      pltpu.sync_copy(dst_hbm.at[pl.ds(row0, STAGE)], dst_v)
      pltpu.async_copy(xaug_hbm.at[src_v.at[0]], gbs[0], sems[0])
      for c in range(STAGE):
        b = c % 2
        if c + 1 < STAGE:
          pltpu.async_copy(xaug_hbm.at[src_v.at[c + 1]], gbs[1 - b],
                           sems[1 - b])
        pltpu.make_async_copy(xaug_hbm.at[src_v.at[c]], gbs[b],
                              sems[b]).wait()
        pltpu.sync_copy(gbs[b], acc.at[dst_v.at[c]], add=True)
      return carry

    lax.fori_loop(0, NSTAGE, stage_body, 0)

    plsc.subcore_barrier()

    pltpu.sync_copy(acc.at[pl.ds(base, ROWS_PER_SUB)],
                    out_hbm.at[cid, pl.ds(base, ROWS_PER_SUB)])

  return k(xaug, src2d, dst2d)


def _tc_combine_body(p_ref, x_ref, wl_ref, wr_ref, b_ref, o_ref):
  s = p_ref[0, :, :IN_CH] + p_ref[1, :, :IN_CH]
  cnt = p_ref[0, :, IN_CH:IN_CH + 1] + p_ref[1, :, IN_CH:IN_CH + 1]
  mean = s / jnp.maximum(cnt, 1.0)
  xb = x_ref[:, :IN_CH]
  dims = (((1,), (1,)), ((), ()))
  o_ref[...] = (
      lax.dot_general(mean, wl_ref[...], dims,
                      preferred_element_type=jnp.float32)
      + b_ref[...]
      + lax.dot_general(xb, wr_ref[...], dims,
                        preferred_element_type=jnp.float32))


def _tc_combine(part, xaug, w_l, w_r, b_l):
  blk = 1264
  grid = NPAD // blk
  return pl.pallas_call(
      _tc_combine_body,
      grid=(grid,),
      in_specs=[
          pl.BlockSpec((NUM_CORES, blk, FEAT), lambda i: (0, i, 0)),
          pl.BlockSpec((blk, FEAT), lambda i: (i, 0)),
          pl.BlockSpec((OUT_CH, IN_CH), lambda i: (0, 0)),
          pl.BlockSpec((OUT_CH, IN_CH), lambda i: (0, 0)),
          pl.BlockSpec((1, OUT_CH), lambda i: (0, 0)),
      ],
      out_specs=pl.BlockSpec((blk, OUT_CH), lambda i: (i, 0)),
      out_shape=jax.ShapeDtypeStruct((NPAD, OUT_CH), jnp.float32),
  )(part, xaug, w_l, w_r, b_l)


def kernel(x, edge_index, W_l, b_l, W_r):
  src = edge_index[0]
  dst = edge_index[1]
  pad = EPAD - E
  src_p = jnp.concatenate([src, jnp.full((pad,), N, dtype=jnp.int32)])
  dst_pad = jnp.arange(pad, dtype=jnp.int32) % N
  dst_p = jnp.concatenate([dst, dst_pad])
  src2d = src_p.reshape(NW * CPW, CHUNK)
  dst2d = dst_p.reshape(NW * CPW, CHUNK)

  xaug = jnp.zeros((NPAD, FEAT), jnp.float32)
  xaug = xaug.at[:N, :IN_CH].set(x)
  xaug = xaug.at[:N, IN_CH].set(1.0)

  part = _sc_segment_sum(xaug, src2d, dst2d)
  out = _tc_combine(part, xaug, W_l, W_r, b_l.reshape(1, OUT_CH))
  return out[:N]


# speedup vs baseline: 11.7358x; 3.3027x over previous
import functools

import jax
import jax.numpy as jnp
from jax import lax
from jax.experimental import pallas as pl
from jax.experimental.pallas import tpu as pltpu
from jax.experimental.pallas import tpu_sc as plsc

N = 10000
E = 320000
IN_CH = 128
OUT_CH = 128

NPAD = 10112
ROWS_PER_SUB = 632
NUM_CORES = 2
NUM_SUBCORES = 16
NW = NUM_CORES * NUM_SUBCORES
CHUNK = 125
CPW = 80
STAGE = 8
NSTAGE = CPW // STAGE
CNT_W = 16


def _sc_segment_sum(x, src2d, dst2d):
  mesh = plsc.VectorSubcoreMesh(
      core_axis_name="c", subcore_axis_name="s",
      num_cores=NUM_CORES, num_subcores=NUM_SUBCORES)

  @functools.partial(
      pl.kernel,
      out_type=(
          jax.ShapeDtypeStruct((NUM_CORES, NPAD, IN_CH), jnp.float32),
          jax.ShapeDtypeStruct((NUM_CORES, NPAD, CNT_W), jnp.float32),
      ),
      mesh=mesh,
      compiler_params=pltpu.CompilerParams(use_tc_tiling_on_sc=False),
      scratch_types=[
          pltpu.VMEM((STAGE, CHUNK), jnp.int32),
          pltpu.VMEM((STAGE, CHUNK), jnp.int32),
          pltpu.VMEM((CHUNK, IN_CH), jnp.float32),
          pltpu.VMEM((CHUNK, IN_CH), jnp.float32),
          pltpu.VMEM((CHUNK, CNT_W), jnp.float32),
          pltpu.VMEM_SHARED((NPAD, IN_CH), jnp.float32),
          pltpu.VMEM_SHARED((NPAD, CNT_W), jnp.float32),
          pltpu.SemaphoreType.DMA,
          pltpu.SemaphoreType.DMA,
      ],
  )
  def k(x_hbm, src_hbm, dst_hbm, out_hbm, cnt_hbm, src_v, dst_v, gb0, gb1,
        ones_v, acc, cacc, sem0, sem1):
    cid = lax.axis_index("c")
    sid = lax.axis_index("s")
    wid = cid * NUM_SUBCORES + sid
    base = sid * ROWS_PER_SUB

    zeros16 = jnp.zeros((16,), jnp.float32)
    ones16 = jnp.ones((16,), jnp.float32)

    def zrow(r, carry):
      for j in range(IN_CH // 16):
        gb0[r, pl.ds(j * 16, 16)] = zeros16
      ones_v[r, pl.ds(0, CNT_W)] = zeros16
      return carry

    lax.fori_loop(0, CHUNK, zrow, 0)
    for t in range(5):
      pltpu.sync_copy(gb0.at[pl.ds(0, 120)],
                      acc.at[pl.ds(base + t * 120, 120)])
      pltpu.sync_copy(ones_v.at[pl.ds(0, 120)],
                      cacc.at[pl.ds(base + t * 120, 120)])
    pltpu.sync_copy(gb0.at[pl.ds(0, 32)], acc.at[pl.ds(base + 600, 32)])
    pltpu.sync_copy(ones_v.at[pl.ds(0, 32)], cacc.at[pl.ds(base + 600, 32)])

    def orow(r, carry):
      ones_v[r, pl.ds(0, CNT_W)] = ones16
      return carry

    lax.fori_loop(0, CHUNK, orow, 0)

    plsc.subcore_barrier()

    gbs = (gb0, gb1)
    sems = (sem0, sem1)

    def stage_body(s, carry):
      row0 = wid * CPW + s * STAGE
      pltpu.sync_copy(src_hbm.at[pl.ds(row0, STAGE)], src_v)
      pltpu.sync_copy(dst_hbm.at[pl.ds(row0, STAGE)], dst_v)
      pltpu.async_copy(x_hbm.at[src_v.at[0]], gbs[0], sems[0])
      for c in range(STAGE):
        b = c % 2
        if c + 1 < STAGE:
          pltpu.async_copy(x_hbm.at[src_v.at[c + 1]], gbs[1 - b],
                           sems[1 - b])
        pltpu.sync_copy(ones_v, cacc.at[dst_v.at[c]], add=True)
        pltpu.make_async_copy(x_hbm.at[src_v.at[c]], gbs[b],
                              sems[b]).wait()
        pltpu.sync_copy(gbs[b], acc.at[dst_v.at[c]], add=True)
      return carry

    lax.fori_loop(0, NSTAGE, stage_body, 0)

    plsc.subcore_barrier()

    pltpu.sync_copy(acc.at[pl.ds(base, ROWS_PER_SUB)],
                    out_hbm.at[cid, pl.ds(base, ROWS_PER_SUB)])
    pltpu.sync_copy(cacc.at[pl.ds(base, ROWS_PER_SUB)],
                    cnt_hbm.at[cid, pl.ds(base, ROWS_PER_SUB)])

  return k(x, src2d, dst2d)


def _tc_combine_body(p_ref, c_ref, x_ref, wl_ref, wr_ref, b_ref, o_ref):
  s = p_ref[0] + p_ref[1]
  cnt = c_ref[0, :, 0:1] + c_ref[1, :, 0:1]
  mean = s / jnp.maximum(cnt, 1.0)
  dims = (((1,), (1,)), ((), ()))
  o_ref[...] = (
      lax.dot_general(mean, wl_ref[...], dims,
                      preferred_element_type=jnp.float32)
      + b_ref[...]
      + lax.dot_general(x_ref[...], wr_ref[...], dims,
                        preferred_element_type=jnp.float32))


def _tc_combine(part, cnt, x, w_l, w_r, b_l):
  blk = 400
  grid = N // blk
  return pl.pallas_call(
      _tc_combine_body,
      grid=(grid,),
      in_specs=[
          pl.BlockSpec((NUM_CORES, blk, IN_CH), lambda i: (0, i, 0)),
          pl.BlockSpec((NUM_CORES, blk, CNT_W), lambda i: (0, i, 0)),
          pl.BlockSpec((blk, IN_CH), lambda i: (i, 0)),
          pl.BlockSpec((OUT_CH, IN_CH), lambda i: (0, 0)),
          pl.BlockSpec((OUT_CH, IN_CH), lambda i: (0, 0)),
          pl.BlockSpec((1, OUT_CH), lambda i: (0, 0)),
      ],
      out_specs=pl.BlockSpec((blk, OUT_CH), lambda i: (i, 0)),
      out_shape=jax.ShapeDtypeStruct((N, OUT_CH), jnp.float32),
  )(part, cnt, x, w_l, w_r, b_l)


def kernel(x, edge_index, W_l, b_l, W_r):
  src2d = edge_index[0].reshape(NW * CPW, CHUNK)
  dst2d = edge_index[1].reshape(NW * CPW, CHUNK)
  part, cnt = _sc_segment_sum(x, src2d, dst2d)
  return _tc_combine(part, cnt, x, W_l, W_r, b_l.reshape(1, OUT_CH))


# speedup vs baseline: 11.7649x; 1.0025x over previous
import functools

import jax
import jax.numpy as jnp
from jax import lax
from jax.experimental import pallas as pl
from jax.experimental.pallas import tpu as pltpu
from jax.experimental.pallas import tpu_sc as plsc

N = 10000
E = 320000
IN_CH = 128
OUT_CH = 128

NPAD = 10112
ROWS_PER_SUB = 632
NUM_CORES = 2
NUM_SUBCORES = 16
NW = NUM_CORES * NUM_SUBCORES
CHUNK = 125
CPW = 80
STAGE = 8
NSTAGE = CPW // STAGE


def _sc_segment_sum(x, src2d, dst2d):
  mesh = plsc.VectorSubcoreMesh(
      core_axis_name="c", subcore_axis_name="s",
      num_cores=NUM_CORES, num_subcores=NUM_SUBCORES)

  @functools.partial(
      pl.kernel,
      out_type=(
          jax.ShapeDtypeStruct((NUM_CORES, NPAD, IN_CH), jnp.float32),
          jax.ShapeDtypeStruct((NW, NPAD), jnp.float32),
      ),
      mesh=mesh,
      compiler_params=pltpu.CompilerParams(use_tc_tiling_on_sc=False,
                                           needs_layout_passes=False),
      scratch_types=[
          pltpu.VMEM((STAGE, CHUNK), jnp.int32),
          pltpu.VMEM((STAGE, CHUNK), jnp.int32),
          pltpu.VMEM((STAGE, CHUNK), jnp.int32),
          pltpu.VMEM((STAGE, CHUNK), jnp.int32),
          pltpu.VMEM((CHUNK, IN_CH), jnp.float32),
          pltpu.VMEM((CHUNK, IN_CH), jnp.float32),
          pltpu.VMEM((NPAD,), jnp.float32),
          pltpu.VMEM_SHARED((NPAD, IN_CH), jnp.float32),
          pltpu.SemaphoreType.DMA,
          pltpu.SemaphoreType.DMA,
          pltpu.SemaphoreType.DMA,
          pltpu.SemaphoreType.DMA,
      ],
  )
  def k(x_hbm, src_hbm, dst_hbm, out_hbm, cnt_hbm, srcA, dstA, srcB, dstB,
        gb0, gb1, hist, acc, sg0, sg1, ss0, ss1):
    cid = lax.axis_index("c")
    sid = lax.axis_index("s")
    wid = cid * NUM_SUBCORES + sid
    base = sid * ROWS_PER_SUB

    zeros16 = jnp.zeros((16,), jnp.float32)
    ones16 = jnp.ones((16,), jnp.float32)
    tail_mask = lax.iota(jnp.int32, 16) >= 3

    def zrow(r, carry):
      for j in range(IN_CH // 16):
        gb0[r, pl.ds(j * 16, 16)] = zeros16
        gb1[r, pl.ds(j * 16, 16)] = zeros16
      return carry

    lax.fori_loop(0, CHUNK, zrow, 0)

    def hrow(r, carry):
      hist[pl.ds(pl.multiple_of(r * 16, 16), 16)] = zeros16
      return carry

    lax.fori_loop(0, NPAD // 16, hrow, 0)

    for t in range(5):
      pltpu.sync_copy(gb0.at[pl.ds(0, 120)],
                      acc.at[pl.ds(base + t * 120, 120)])
    pltpu.sync_copy(gb0.at[pl.ds(0, 32)], acc.at[pl.ds(base + 600, 32)])

    pltpu.sync_copy(src_hbm.at[pl.ds(wid * CPW, STAGE)], srcA)
    pltpu.sync_copy(dst_hbm.at[pl.ds(wid * CPW, STAGE)], dstA)

    plsc.subcore_barrier()

    pltpu.async_copy(gb1, acc.at[dstA.at[0]], ss1, add=True)
    pltpu.async_copy(x_hbm.at[srcA.at[0]], gb0, sg0)

    gbs = (gb0, gb1)
    sgs = (sg0, sg1)
    sss = (ss0, ss1)

    def hist_chunk(dst_ref, c):
      for j in range(7):
        idx = dst_ref[c, pl.ds(j * 16, 16)]
        plsc.addupdate_scatter(hist, [idx], ones16)
      idx = dst_ref[c, pl.ds(109, 16)]
      plsc.addupdate_scatter(hist, [idx], ones16, mask=tail_mask)

    def two_stages(i, carry):
      for half, (srcP, dstP, srcQ, dstQ) in enumerate(
          ((srcA, dstA, srcB, dstB), (srcB, dstB, srcA, dstA))):
        s = 2 * i + half
        for c in range(STAGE):
          b = c % 2
          pltpu.make_async_copy(x_hbm.at[srcP.at[c]], gbs[b], sgs[b]).wait()
          pltpu.async_copy(gbs[b], acc.at[dstP.at[c]], sss[b], add=True)
          hist_chunk(dstP, c)
          if c == 2:
            @pl.when(s < NSTAGE - 1)
            def _():
              row0 = wid * CPW + (s + 1) * STAGE
              pltpu.sync_copy(src_hbm.at[pl.ds(row0, STAGE)], srcQ)
              pltpu.sync_copy(dst_hbm.at[pl.ds(row0, STAGE)], dstQ)
          pltpu.make_async_copy(gbs[1 - b], acc.at[dstP.at[c]],
                                sss[1 - b]).wait()
          if c + 1 < STAGE:
            pltpu.async_copy(x_hbm.at[srcP.at[c + 1]], gbs[1 - b],
                             sgs[1 - b])
          else:
            @pl.when(s < NSTAGE - 1)
            def _():
              pltpu.async_copy(x_hbm.at[srcQ.at[0]], gbs[1 - b],
                               sgs[1 - b])
      return carry

    lax.fori_loop(0, NSTAGE // 2, two_stages, 0)

    pltpu.make_async_copy(gbs[1], acc.at[dstB.at[STAGE - 1]], ss1).wait()

    plsc.subcore_barrier()

    pltpu.sync_copy(acc.at[pl.ds(base, ROWS_PER_SUB)],
                    out_hbm.at[cid, pl.ds(base, ROWS_PER_SUB)])
    pltpu.sync_copy(hist, cnt_hbm.at[wid])

  return k(x, src2d, dst2d)


def _tc_combine_body(p_ref, c_ref, x_ref, wl_ref, wr_ref, b_ref, o_ref):
  s = p_ref[0] + p_ref[1]
  cnt = lax.dot_general(c_ref[...], jnp.ones((NW, 1), jnp.float32),
                        (((0,), (0,)), ((), ())),
                        preferred_element_type=jnp.float32)
  mean = s / jnp.maximum(cnt, 1.0)
  dims = (((1,), (1,)), ((), ()))
  o_ref[...] = (
      lax.dot_general(mean, wl_ref[...], dims,
                      preferred_element_type=jnp.float32)
      + b_ref[...]
      + lax.dot_general(x_ref[...], wr_ref[...], dims,
                        preferred_element_type=jnp.float32))


def _tc_combine(part, cnt, x, w_l, w_r, b_l):
  blk = 1280
  grid = pl.cdiv(N, blk)
  return pl.pallas_call(
      _tc_combine_body,
      grid=(grid,),
      in_specs=[
          pl.BlockSpec((NUM_CORES, blk, IN_CH), lambda i: (0, i, 0)),
          pl.BlockSpec((NW, blk), lambda i: (0, i)),
          pl.BlockSpec((blk, IN_CH), lambda i: (i, 0)),
          pl.BlockSpec((OUT_CH, IN_CH), lambda i: (0, 0)),
          pl.BlockSpec((OUT_CH, IN_CH), lambda i: (0, 0)),
          pl.BlockSpec((1, OUT_CH), lambda i: (0, 0)),
      ],
      out_specs=pl.BlockSpec((blk, OUT_CH), lambda i: (i, 0)),
      out_shape=jax.ShapeDtypeStruct((N, OUT_CH), jnp.float32),
  )(part, cnt, x, w_l, w_r, b_l)


def kernel(x, edge_index, W_l, b_l, W_r):
  src2d = edge_index[0].reshape(NW * CPW, CHUNK)
  dst2d = edge_index[1].reshape(NW * CPW, CHUNK)
  part, cnt = _sc_segment_sum(x, src2d, dst2d)
  return _tc_combine(part, cnt, x, W_l, W_r, b_l.reshape(1, OUT_CH))


# speedup vs baseline: 12.0061x; 1.0205x over previous
import functools

import jax
import jax.numpy as jnp
from jax import lax
from jax.experimental import pallas as pl
from jax.experimental.pallas import tpu as pltpu
from jax.experimental.pallas import tpu_sc as plsc

N = 10000
E = 320000
IN_CH = 128
OUT_CH = 128

NPAD = 10112
ROWS_PER_SUB = 632
NUM_CORES = 2
NUM_SUBCORES = 16
NW = NUM_CORES * NUM_SUBCORES
CHUNK = 50
CPW = 200
HALF = 100
NBUF = 3
GROUPS = 32


def _sc_segment_sum(x, src2d, dst2d):
  mesh = plsc.VectorSubcoreMesh(
      core_axis_name="c", subcore_axis_name="s",
      num_cores=NUM_CORES, num_subcores=NUM_SUBCORES)

  @functools.partial(
      pl.kernel,
      out_type=(
          jax.ShapeDtypeStruct((NUM_CORES, NPAD, IN_CH), jnp.float32),
          jax.ShapeDtypeStruct((NW, NPAD), jnp.float32),
      ),
      mesh=mesh,
      compiler_params=pltpu.CompilerParams(use_tc_tiling_on_sc=False,
                                           needs_layout_passes=False),
      scratch_types=[
          pltpu.VMEM((HALF, CHUNK), jnp.int32),
          pltpu.VMEM((HALF, CHUNK), jnp.int32),
          pltpu.VMEM((CHUNK, IN_CH), jnp.float32),
          pltpu.VMEM((CHUNK, IN_CH), jnp.float32),
          pltpu.VMEM((CHUNK, IN_CH), jnp.float32),
          pltpu.VMEM((NPAD,), jnp.float32),
          pltpu.VMEM_SHARED((NPAD, IN_CH), jnp.float32),
          pltpu.SemaphoreType.DMA,
          pltpu.SemaphoreType.DMA,
          pltpu.SemaphoreType.DMA,
          pltpu.SemaphoreType.DMA,
          pltpu.SemaphoreType.DMA,
          pltpu.SemaphoreType.DMA,
      ],
  )
  def k(x_hbm, src_hbm, dst_hbm, out_hbm, cnt_hbm, src_v, dst_v,
        gb0, gb1, gb2, hist, acc, sg0, sg1, sg2, ss0, ss1, ss2):
    cid = lax.axis_index("c")
    sid = lax.axis_index("s")
    wid = cid * NUM_SUBCORES + sid
    base = sid * ROWS_PER_SUB

    zeros16 = jnp.zeros((16,), jnp.float32)
    ones16 = jnp.ones((16,), jnp.float32)
    tail_mask = lax.iota(jnp.int32, 16) >= 14

    gbs = (gb0, gb1, gb2)
    sgs = (sg0, sg1, sg2)
    sss = (ss0, ss1, ss2)

    def zrow(r, carry):
      for j in range(IN_CH // 16):
        gb0[r, pl.ds(j * 16, 16)] = zeros16
        gb2[r, pl.ds(j * 16, 16)] = zeros16
      return carry

    lax.fori_loop(0, CHUNK, zrow, 0)

    def hrow(r, carry):
      hist[pl.ds(pl.multiple_of(r * 16, 16), 16)] = zeros16
      return carry

    lax.fori_loop(0, NPAD // 16, hrow, 0)

    for t in range(13):
      pltpu.sync_copy(gb0.at[pl.ds(0, 48)],
                      acc.at[pl.ds(base + t * 48, 48)])
    pltpu.sync_copy(gb0.at[pl.ds(0, 8)], acc.at[pl.ds(base + 624, 8)])

    plsc.subcore_barrier()

    def hist_chunk(g):
      for j in range(3):
        idx = dst_v[g, pl.ds(j * 16, 16)]
        plsc.addupdate_scatter(hist, [idx], ones16)
      idx = dst_v[g, pl.ds(34, 16)]
      plsc.addupdate_scatter(hist, [idx], ones16, mask=tail_mask)

    def chunk_body(g, b, fire_next, wait_prev=True):
      pltpu.make_async_copy(x_hbm.at[src_v.at[g]], gbs[b], sgs[b]).wait()
      pltpu.async_copy(gbs[b], acc.at[dst_v.at[g]], sss[b], add=True)
      hist_chunk(g)
      nb = (b + 2) % NBUF
      if wait_prev:
        pltpu.make_async_copy(gbs[nb], acc.at[dst_v.at[g]], sss[nb]).wait()
      if fire_next:
        pltpu.async_copy(x_hbm.at[src_v.at[g + 2]], gbs[nb], sgs[nb])

    def group(i, carry):
      for u in range(NBUF):
        chunk_body(NBUF * i + u, u, True)
      return carry

    for phase in range(2):
      pltpu.sync_copy(src_hbm.at[pl.ds(wid * CPW + phase * HALF, HALF)],
                      src_v)
      pltpu.sync_copy(dst_hbm.at[pl.ds(wid * CPW + phase * HALF, HALF)],
                      dst_v)
      if phase == 0:
        pltpu.async_copy(gb2, acc.at[dst_v.at[0]], ss2, add=True)
      pltpu.async_copy(x_hbm.at[src_v.at[0]], gb0, sg0)
      pltpu.async_copy(x_hbm.at[src_v.at[1]], gb1, sg1)
      if phase == 0:
        lax.fori_loop(0, GROUPS, group, 0)
      else:
        chunk_body(0, 0, True, wait_prev=False)
        chunk_body(1, 1, True)
        chunk_body(2, 2, True)

        def group1(i, carry):
          for u in range(NBUF):
            chunk_body(NBUF * (i + 1) + u, u, True)
          return carry

        lax.fori_loop(0, GROUPS - 1, group1, 0)
      chunk_body(HALF - 4, 0, True)
      chunk_body(HALF - 3, 1, True)
      chunk_body(HALF - 2, 2, False)
      chunk_body(HALF - 1, 0, False)
      pltpu.make_async_copy(gbs[0], acc.at[dst_v.at[HALF - 1]], ss0).wait()

    plsc.subcore_barrier()

    pltpu.sync_copy(acc.at[pl.ds(base, ROWS_PER_SUB)],
                    out_hbm.at[cid, pl.ds(base, ROWS_PER_SUB)])
    pltpu.sync_copy(hist, cnt_hbm.at[wid])

  return k(x, src2d, dst2d)


def _tc_combine_body(p_ref, c_ref, x_ref, wl_ref, wr_ref, b_ref, o_ref):
  s = p_ref[0] + p_ref[1]
  cnt = lax.dot_general(c_ref[...], jnp.ones((NW, 1), jnp.float32),
                        (((0,), (0,)), ((), ())),
                        preferred_element_type=jnp.float32)
  mean = s / jnp.maximum(cnt, 1.0)
  dims = (((1,), (1,)), ((), ()))
  o_ref[...] = (
      lax.dot_general(mean, wl_ref[...], dims,
                      preferred_element_type=jnp.float32)
      + b_ref[...]
      + lax.dot_general(x_ref[...], wr_ref[...], dims,
                        preferred_element_type=jnp.float32))


def _tc_combine(part, cnt, x, w_l, w_r, b_l):
  blk = 1280
  grid = pl.cdiv(N, blk)
  return pl.pallas_call(
      _tc_combine_body,
      grid=(grid,),
      in_specs=[
          pl.BlockSpec((NUM_CORES, blk, IN_CH), lambda i: (0, i, 0)),
          pl.BlockSpec((NW, blk), lambda i: (0, i)),
          pl.BlockSpec((blk, IN_CH), lambda i: (i, 0)),
          pl.BlockSpec((OUT_CH, IN_CH), lambda i: (0, 0)),
          pl.BlockSpec((OUT_CH, IN_CH), lambda i: (0, 0)),
          pl.BlockSpec((1, OUT_CH), lambda i: (0, 0)),
      ],
      out_specs=pl.BlockSpec((blk, OUT_CH), lambda i: (i, 0)),
      out_shape=jax.ShapeDtypeStruct((N, OUT_CH), jnp.float32),
  )(part, cnt, x, w_l, w_r, b_l)


def kernel(x, edge_index, W_l, b_l, W_r):
  src2d = edge_index[0].reshape(NW * CPW, CHUNK)
  dst2d = edge_index[1].reshape(NW * CPW, CHUNK)
  part, cnt = _sc_segment_sum(x, src2d, dst2d)
  return _tc_combine(part, cnt, x, W_l, W_r, b_l.reshape(1, OUT_CH))


# speedup vs baseline: 14.4568x; 1.2041x over previous
import functools

import jax
import jax.numpy as jnp
from jax import lax
from jax.experimental import pallas as pl
from jax.experimental.pallas import tpu as pltpu
from jax.experimental.pallas import tpu_sc as plsc

N = 10000
E = 320000
IN_CH = 128
OUT_CH = 128

NPAD = 10112
ROWS_PER_SUB = 632
NUM_CORES = 2
NUM_SUBCORES = 16
NW = NUM_CORES * NUM_SUBCORES
CHUNK = 125
CPW = 80
WIN = 16
GROUPS = CPW // WIN


def _sc_segment_sum(x, src2d, dst2d):
  mesh = plsc.VectorSubcoreMesh(
      core_axis_name="c", subcore_axis_name="s",
      num_cores=NUM_CORES, num_subcores=NUM_SUBCORES)

  @functools.partial(
      pl.kernel,
      out_type=(
          jax.ShapeDtypeStruct((NUM_CORES, NPAD, IN_CH), jnp.float32),
          jax.ShapeDtypeStruct((NW, NPAD), jnp.float32),
      ),
      mesh=mesh,
      compiler_params=pltpu.CompilerParams(use_tc_tiling_on_sc=False,
                                           needs_layout_passes=False),
      scratch_types=[
          pltpu.VMEM((WIN, CHUNK), jnp.int32),
          pltpu.VMEM((WIN, CHUNK), jnp.int32),
          pltpu.VMEM((CHUNK, IN_CH), jnp.float32),
          pltpu.VMEM((CHUNK, IN_CH), jnp.float32),
          pltpu.VMEM((NPAD,), jnp.float32),
          pltpu.VMEM_SHARED((NPAD, IN_CH), jnp.float32),
          pltpu.SemaphoreType.DMA,
          pltpu.SemaphoreType.DMA,
          pltpu.SemaphoreType.DMA,
          pltpu.SemaphoreType.DMA,
          pltpu.SemaphoreType.DMA,
      ],
  )
  def k(x_hbm, src_hbm, dst_hbm, out_hbm, cnt_hbm, src_v, dst_v,
        gb0, gb1, hist, acc, sg0, sg1, ss0, ss1, sr):
    cid = lax.axis_index("c")
    sid = lax.axis_index("s")
    wid = cid * NUM_SUBCORES + sid
    base = sid * ROWS_PER_SUB

    zeros16 = jnp.zeros((16,), jnp.float32)
    ones16 = jnp.ones((16,), jnp.float32)
    tail_mask = lax.iota(jnp.int32, 16) >= 3

    gbs = (gb0, gb1)
    sgs = (sg0, sg1)
    sss = (ss0, ss1)

    def zrow(r, carry):
      for j in range(IN_CH // 16):
        gb0[r, pl.ds(j * 16, 16)] = zeros16
      return carry

    lax.fori_loop(0, CHUNK, zrow, 0)

    def hrow(r, carry):
      hist[pl.ds(pl.multiple_of(r * 16, 16), 16)] = zeros16
      return carry

    lax.fori_loop(0, NPAD // 16, hrow, 0)

    for t in range(5):
      pltpu.sync_copy(gb0.at[pl.ds(0, 120)],
                      acc.at[pl.ds(base + t * 120, 120)])
    pltpu.sync_copy(gb0.at[pl.ds(0, 32)], acc.at[pl.ds(base + 600, 32)])

    pltpu.sync_copy(src_hbm.at[pl.ds(wid * CPW, WIN)], src_v)
    pltpu.sync_copy(dst_hbm.at[pl.ds(wid * CPW, WIN)], dst_v)

    plsc.subcore_barrier()

    pltpu.async_copy(x_hbm.at[src_v.at[0]], gb0, sg0)
    pltpu.async_copy(x_hbm.at[src_v.at[1]], gb1, sg1)

    def hist_chunk(c):
      for j in range(7):
        idx = dst_v[c, pl.ds(j * 16, 16)]
        plsc.addupdate_scatter(hist, [idx], ones16)
      idx = dst_v[c, pl.ds(109, 16)]
      plsc.addupdate_scatter(hist, [idx], ones16, mask=tail_mask)

    def group(i, carry):
      last = i == GROUPS - 1
      for c in range(WIN):
        b = c % 2
        pltpu.make_async_copy(x_hbm.at[src_v.at[c]], gbs[b], sgs[b]).wait()
        pltpu.async_copy(gbs[b], acc.at[dst_v.at[c]], sss[b], add=True)
        if c == 0:
          row0 = wid * CPW + i * WIN + 8
          pltpu.async_copy(src_hbm.at[pl.ds(row0, 8)],
                           src_v.at[pl.ds(8, 8)], sr)
          pltpu.async_copy(dst_hbm.at[pl.ds(row0, 8)],
                           dst_v.at[pl.ds(8, 8)], sr)
        if c == 8:
          @pl.when(jnp.logical_not(last))
          def _():
            row0 = wid * CPW + (i + 1) * WIN
            pltpu.async_copy(src_hbm.at[pl.ds(row0, 8)],
                             src_v.at[pl.ds(0, 8)], sr)
            pltpu.async_copy(dst_hbm.at[pl.ds(row0, 8)],
                             dst_v.at[pl.ds(0, 8)], sr)
        hist_chunk(c)
        if c == 5:
          pltpu.make_async_copy(src_hbm.at[pl.ds(0, 8)],
                                src_v.at[pl.ds(8, 8)], sr).wait()
          pltpu.make_async_copy(dst_hbm.at[pl.ds(0, 8)],
                                dst_v.at[pl.ds(8, 8)], sr).wait()
        if c == 13:
          @pl.when(jnp.logical_not(last))
          def _():
            pltpu.make_async_copy(src_hbm.at[pl.ds(0, 8)],
                                  src_v.at[pl.ds(0, 8)], sr).wait()
            pltpu.make_async_copy(dst_hbm.at[pl.ds(0, 8)],
                                  dst_v.at[pl.ds(0, 8)], sr).wait()
        pltpu.make_async_copy(gbs[b], acc.at[dst_v.at[c]], sss[b]).wait()
        if c + 2 < WIN:
          pltpu.async_copy(x_hbm.at[src_v.at[c + 2]], gbs[b], sgs[b])
        else:
          @pl.when(jnp.logical_not(last))
          def _():
            pltpu.async_copy(x_hbm.at[src_v.at[c + 2 - WIN]], gbs[b],
                             sgs[b])
      return carry

    lax.fori_loop(0, GROUPS, group, 0)

    plsc.subcore_barrier()

    pltpu.sync_copy(acc.at[pl.ds(base, ROWS_PER_SUB)],
                    out_hbm.at[cid, pl.ds(base, ROWS_PER_SUB)])
    pltpu.sync_copy(hist, cnt_hbm.at[wid])

  return k(x, src2d, dst2d)


def _tc_combine_body(p_ref, c_ref, x_ref, wl_ref, wr_ref, b_ref, o_ref):
  s = p_ref[0] + p_ref[1]
  cnt = lax.dot_general(c_ref[...], jnp.ones((NW, 1), jnp.float32),
                        (((0,), (0,)), ((), ())),
                        preferred_element_type=jnp.float32)
  mean = s / jnp.maximum(cnt, 1.0)
  dims = (((1,), (1,)), ((), ()))
  o_ref[...] = (
      lax.dot_general(mean, wl_ref[...], dims,
                      preferred_element_type=jnp.float32)
      + b_ref[...]
      + lax.dot_general(x_ref[...], wr_ref[...], dims,
                        preferred_element_type=jnp.float32))


def _tc_combine(part, cnt, x, w_l, w_r, b_l):
  blk = 1280
  grid = pl.cdiv(N, blk)
  return pl.pallas_call(
      _tc_combine_body,
      grid=(grid,),
      in_specs=[
          pl.BlockSpec((NUM_CORES, blk, IN_CH), lambda i: (0, i, 0)),
          pl.BlockSpec((NW, blk), lambda i: (0, i)),
          pl.BlockSpec((blk, IN_CH), lambda i: (i, 0)),
          pl.BlockSpec((OUT_CH, IN_CH), lambda i: (0, 0)),
          pl.BlockSpec((OUT_CH, IN_CH), lambda i: (0, 0)),
          pl.BlockSpec((1, OUT_CH), lambda i: (0, 0)),
      ],
      out_specs=pl.BlockSpec((blk, OUT_CH), lambda i: (i, 0)),
      out_shape=jax.ShapeDtypeStruct((N, OUT_CH), jnp.float32),
  )(part, cnt, x, w_l, w_r, b_l)


def kernel(x, edge_index, W_l, b_l, W_r):
  src2d = edge_index[0].reshape(NW * CPW, CHUNK)
  dst2d = edge_index[1].reshape(NW * CPW, CHUNK)
  part, cnt = _sc_segment_sum(x, src2d, dst2d)
  return _tc_combine(part, cnt, x, W_l, W_r, b_l.reshape(1, OUT_CH))


# speedup vs baseline: 15.6711x; 1.0840x over previous
import functools

import jax
import jax.numpy as jnp
from jax import lax
from jax.experimental import pallas as pl
from jax.experimental.pallas import tpu as pltpu
from jax.experimental.pallas import tpu_sc as plsc

N = 10000
E = 320000
IN_CH = 128
OUT_CH = 128

NPAD = 10112
ROWS_PER_SUB = 632
NUM_CORES = 2
NUM_SUBCORES = 16
NW = NUM_CORES * NUM_SUBCORES
CHUNK = 125
CPW = 80
WIN = 16
GROUPS = CPW // WIN


def _sc_segment_sum(x, ei3):
  mesh = plsc.VectorSubcoreMesh(
      core_axis_name="c", subcore_axis_name="s",
      num_cores=NUM_CORES, num_subcores=NUM_SUBCORES)

  @functools.partial(
      pl.kernel,
      out_type=(
          jax.ShapeDtypeStruct((NUM_CORES, NPAD, IN_CH), jnp.float32),
          jax.ShapeDtypeStruct((NW, NPAD), jnp.float32),
      ),
      mesh=mesh,
      compiler_params=pltpu.CompilerParams(use_tc_tiling_on_sc=False,
                                           needs_layout_passes=False),
      scratch_types=[
          pltpu.VMEM((WIN, CHUNK), jnp.int32),
          pltpu.VMEM((WIN, CHUNK), jnp.int32),
          pltpu.VMEM((CHUNK, IN_CH), jnp.float32),
          pltpu.VMEM((CHUNK, IN_CH), jnp.float32),
          pltpu.VMEM((NPAD,), jnp.float32),
          pltpu.VMEM_SHARED((NPAD, IN_CH), jnp.float32),
          pltpu.SemaphoreType.DMA,
          pltpu.SemaphoreType.DMA,
          pltpu.SemaphoreType.DMA,
          pltpu.SemaphoreType.DMA,
          pltpu.SemaphoreType.DMA,
      ],
  )
  def k(x_hbm, ei_hbm, out_hbm, cnt_hbm, src_v, dst_v,
        gb0, gb1, hist, acc, sg0, sg1, ss0, ss1, sr):
    src_hbm = ei_hbm.at[0]
    dst_hbm = ei_hbm.at[1]
    cid = lax.axis_index("c")
    sid = lax.axis_index("s")
    wid = cid * NUM_SUBCORES + sid
    base = sid * ROWS_PER_SUB

    zeros16 = jnp.zeros((16,), jnp.float32)
    ones16 = jnp.ones((16,), jnp.float32)
    tail_mask = lax.iota(jnp.int32, 16) >= 3

    gbs = (gb0, gb1)
    sgs = (sg0, sg1)
    sss = (ss0, ss1)

    def zrow(r, carry):
      for j in range(IN_CH // 16):
        gb0[r, pl.ds(j * 16, 16)] = zeros16
      return carry

    lax.fori_loop(0, CHUNK, zrow, 0)

    def hrow(r, carry):
      hist[pl.ds(pl.multiple_of(r * 16, 16), 16)] = zeros16
      return carry

    lax.fori_loop(0, NPAD // 16, hrow, 0)

    for t in range(5):
      pltpu.sync_copy(gb0.at[pl.ds(0, 120)],
                      acc.at[pl.ds(base + t * 120, 120)])
    pltpu.sync_copy(gb0.at[pl.ds(0, 32)], acc.at[pl.ds(base + 600, 32)])

    pltpu.sync_copy(src_hbm.at[pl.ds(wid * CPW, WIN)], src_v)
    pltpu.sync_copy(dst_hbm.at[pl.ds(wid * CPW, WIN)], dst_v)

    plsc.subcore_barrier()

    pltpu.async_copy(x_hbm.at[src_v.at[0]], gb0, sg0)
    pltpu.async_copy(x_hbm.at[src_v.at[1]], gb1, sg1)

    def hist_chunk(c):
      for j in range(7):
        idx = dst_v[c, pl.ds(j * 16, 16)]
        plsc.addupdate_scatter(hist, [idx], ones16)
      idx = dst_v[c, pl.ds(109, 16)]
      plsc.addupdate_scatter(hist, [idx], ones16, mask=tail_mask)

    def group(i, carry):
      last = i == GROUPS - 1
      for c in range(WIN):
        b = c % 2
        pltpu.make_async_copy(x_hbm.at[src_v.at[c]], gbs[b], sgs[b]).wait()
        pltpu.async_copy(gbs[b], acc.at[dst_v.at[c]], sss[b], add=True)
        if c == 0:
          row0 = wid * CPW + i * WIN + 8
          pltpu.async_copy(src_hbm.at[pl.ds(row0, 8)],
                           src_v.at[pl.ds(8, 8)], sr)
          pltpu.async_copy(dst_hbm.at[pl.ds(row0, 8)],
                           dst_v.at[pl.ds(8, 8)], sr)
        if c == 8:
          @pl.when(jnp.logical_not(last))
          def _():
            row0 = wid * CPW + (i + 1) * WIN
            pltpu.async_copy(src_hbm.at[pl.ds(row0, 8)],
                             src_v.at[pl.ds(0, 8)], sr)
            pltpu.async_copy(dst_hbm.at[pl.ds(row0, 8)],
                             dst_v.at[pl.ds(0, 8)], sr)
        hist_chunk(c)
        if c == 5:
          pltpu.make_async_copy(src_hbm.at[pl.ds(0, 8)],
                                src_v.at[pl.ds(8, 8)], sr).wait()
          pltpu.make_async_copy(dst_hbm.at[pl.ds(0, 8)],
                                dst_v.at[pl.ds(8, 8)], sr).wait()
        if c == 13:
          @pl.when(jnp.logical_not(last))
          def _():
            pltpu.make_async_copy(src_hbm.at[pl.ds(0, 8)],
                                  src_v.at[pl.ds(0, 8)], sr).wait()
            pltpu.make_async_copy(dst_hbm.at[pl.ds(0, 8)],
                                  dst_v.at[pl.ds(0, 8)], sr).wait()
        pltpu.make_async_copy(gbs[b], acc.at[dst_v.at[c]], sss[b]).wait()
        if c + 2 < WIN:
          pltpu.async_copy(x_hbm.at[src_v.at[c + 2]], gbs[b], sgs[b])
        else:
          @pl.when(jnp.logical_not(last))
          def _():
            pltpu.async_copy(x_hbm.at[src_v.at[c + 2 - WIN]], gbs[b],
                             sgs[b])
      return carry

    lax.fori_loop(0, GROUPS, group, 0)

    plsc.subcore_barrier()

    pltpu.sync_copy(acc.at[pl.ds(base, ROWS_PER_SUB)],
                    out_hbm.at[cid, pl.ds(base, ROWS_PER_SUB)])
    pltpu.sync_copy(hist, cnt_hbm.at[wid])

  return k(x, ei3)


def _tc_combine_body(p_ref, c_ref, x_ref, wl_ref, wr_ref, b_ref, o_ref):
  s = p_ref[0] + p_ref[1]
  cnt = lax.dot_general(c_ref[...], jnp.ones((NW, 1), jnp.float32),
                        (((0,), (0,)), ((), ())),
                        preferred_element_type=jnp.float32)
  mean = s / jnp.maximum(cnt, 1.0)
  dims = (((1,), (1,)), ((), ()))
  o_ref[...] = (
      lax.dot_general(mean, wl_ref[...], dims,
                      preferred_element_type=jnp.float32)
      + b_ref[...]
      + lax.dot_general(x_ref[...], wr_ref[...], dims,
                        preferred_element_type=jnp.float32))


def _tc_combine(part, cnt, x, w_l, w_r, b_l):
  blk = 2048
  grid = pl.cdiv(N, blk)
  return pl.pallas_call(
      _tc_combine_body,
      grid=(grid,),
      in_specs=[
          pl.BlockSpec((NUM_CORES, blk, IN_CH), lambda i: (0, i, 0)),
          pl.BlockSpec((NW, blk), lambda i: (0, i)),
          pl.BlockSpec((blk, IN_CH), lambda i: (i, 0)),
          pl.BlockSpec((OUT_CH, IN_CH), lambda i: (0, 0)),
          pl.BlockSpec((OUT_CH, IN_CH), lambda i: (0, 0)),
          pl.BlockSpec((1, OUT_CH), lambda i: (0, 0)),
      ],
      out_specs=pl.BlockSpec((blk, OUT_CH), lambda i: (i, 0)),
      out_shape=jax.ShapeDtypeStruct((N, OUT_CH), jnp.float32),
  )(part, cnt, x, w_l, w_r, b_l)


def kernel(x, edge_index, W_l, b_l, W_r):
  ei3 = edge_index.reshape(2, NW * CPW, CHUNK)
  part, cnt = _sc_segment_sum(x, ei3)
  return _tc_combine(part, cnt, x, W_l, W_r, b_l.reshape(1, OUT_CH))


# speedup vs baseline: 15.7107x; 1.0025x over previous
import functools

import jax
import jax.numpy as jnp
from jax import lax
from jax.experimental import pallas as pl
from jax.experimental.pallas import tpu as pltpu
from jax.experimental.pallas import tpu_sc as plsc

N = 10000
E = 320000
IN_CH = 128
OUT_CH = 128

NPAD = 10112
ROWS_PER_SUB = 632
NUM_CORES = 2
NUM_SUBCORES = 16
NW = NUM_CORES * NUM_SUBCORES
CHUNK = 125
CPW = 80
WIN = 16
GROUPS = CPW // WIN


def _sc_segment_sum(x, ei3):
  mesh = plsc.VectorSubcoreMesh(
      core_axis_name="c", subcore_axis_name="s",
      num_cores=NUM_CORES, num_subcores=NUM_SUBCORES)

  @functools.partial(
      pl.kernel,
      out_type=(
          jax.ShapeDtypeStruct((NUM_CORES, NPAD, IN_CH), jnp.float32),
          jax.ShapeDtypeStruct((NW, NPAD), jnp.float32),
      ),
      mesh=mesh,
      compiler_params=pltpu.CompilerParams(use_tc_tiling_on_sc=False,
                                           needs_layout_passes=False),
      scratch_types=[
          pltpu.VMEM((WIN, CHUNK), jnp.int32),
          pltpu.VMEM((WIN, CHUNK), jnp.int32),
          pltpu.VMEM((CHUNK, IN_CH), jnp.float32),
          pltpu.VMEM((CHUNK, IN_CH), jnp.float32),
          pltpu.VMEM((NPAD,), jnp.float32),
          pltpu.VMEM_SHARED((NPAD, IN_CH), jnp.float32),
          pltpu.SemaphoreType.DMA,
          pltpu.SemaphoreType.DMA,
          pltpu.SemaphoreType.DMA,
          pltpu.SemaphoreType.DMA,
          pltpu.SemaphoreType.DMA,
      ],
  )
  def k(x_hbm, ei_hbm, out_hbm, cnt_hbm, src_v, dst_v,
        gb0, gb1, hist, acc, sg0, sg1, ss0, ss1, sr):
    src_hbm = ei_hbm.at[0]
    dst_hbm = ei_hbm.at[1]
    cid = lax.axis_index("c")
    sid = lax.axis_index("s")
    wid = cid * NUM_SUBCORES + sid
    base = sid * ROWS_PER_SUB

    zeros16 = jnp.zeros((16,), jnp.float32)
    ones16 = jnp.ones((16,), jnp.float32)
    tail_mask = lax.iota(jnp.int32, 16) >= 3

    gbs = (gb0, gb1)
    sgs = (sg0, sg1)
    sss = (ss0, ss1)

    def zrow(r, carry):
      for j in range(IN_CH // 16):
        gb0[r, pl.ds(j * 16, 16)] = zeros16
      return carry

    lax.fori_loop(0, CHUNK, zrow, 0)

    def hrow(r, carry):
      hist[pl.ds(pl.multiple_of(r * 16, 16), 16)] = zeros16
      return carry

    lax.fori_loop(0, NPAD // 16, hrow, 0)

    for t in range(5):
      pltpu.sync_copy(gb0.at[pl.ds(0, 120)],
                      acc.at[pl.ds(base + t * 120, 120)])
    pltpu.sync_copy(gb0.at[pl.ds(0, 32)], acc.at[pl.ds(base + 600, 32)])

    pltpu.sync_copy(src_hbm.at[pl.ds(wid * CPW, WIN)], src_v)
    pltpu.sync_copy(dst_hbm.at[pl.ds(wid * CPW, WIN)], dst_v)

    plsc.subcore_barrier()

    def fire_gather(c, b):
      pltpu.async_copy(x_hbm.at[src_v.at[c, pl.ds(0, 64)]],
                       gbs[b].at[pl.ds(0, 64)], sgs[b])
      pltpu.async_copy(x_hbm.at[src_v.at[c, pl.ds(64, 61)]],
                       gbs[b].at[pl.ds(64, 61)], sgs[b])

    def wait_gather(c, b):
      pltpu.make_async_copy(x_hbm.at[src_v.at[c, pl.ds(0, 64)]],
                            gbs[b].at[pl.ds(0, 64)], sgs[b]).wait()
      pltpu.make_async_copy(x_hbm.at[src_v.at[c, pl.ds(64, 61)]],
                            gbs[b].at[pl.ds(64, 61)], sgs[b]).wait()

    fire_gather(0, 0)
    fire_gather(1, 1)

    def hist_chunk(c):
      for j in range(7):
        idx = dst_v[c, pl.ds(j * 16, 16)]
        plsc.addupdate_scatter(hist, [idx], ones16)
      idx = dst_v[c, pl.ds(109, 16)]
      plsc.addupdate_scatter(hist, [idx], ones16, mask=tail_mask)

    def group(i, carry):
      last = i == GROUPS - 1
      for c in range(WIN):
        b = c % 2
        wait_gather(c, b)
        pltpu.async_copy(gbs[b], acc.at[dst_v.at[c]], sss[b], add=True)
        if c == 0:
          row0 = wid * CPW + i * WIN + 8
          pltpu.async_copy(src_hbm.at[pl.ds(row0, 8)],
                           src_v.at[pl.ds(8, 8)], sr)
          pltpu.async_copy(dst_hbm.at[pl.ds(row0, 8)],
                           dst_v.at[pl.ds(8, 8)], sr)
        if c == 8:
          @pl.when(jnp.logical_not(last))
          def _():
            row0 = wid * CPW + (i + 1) * WIN
            pltpu.async_copy(src_hbm.at[pl.ds(row0, 8)],
                             src_v.at[pl.ds(0, 8)], sr)
            pltpu.async_copy(dst_hbm.at[pl.ds(row0, 8)],
                             dst_v.at[pl.ds(0, 8)], sr)
        hist_chunk(c)
        if c == 5:
          pltpu.make_async_copy(src_hbm.at[pl.ds(0, 8)],
                                src_v.at[pl.ds(8, 8)], sr).wait()
          pltpu.make_async_copy(dst_hbm.at[pl.ds(0, 8)],
                                dst_v.at[pl.ds(8, 8)], sr).wait()
        if c == 13:
          @pl.when(jnp.logical_not(last))
          def _():
            pltpu.make_async_copy(src_hbm.at[pl.ds(0, 8)],
                                  src_v.at[pl.ds(0, 8)], sr).wait()
            pltpu.make_async_copy(dst_hbm.at[pl.ds(0, 8)],
                                  dst_v.at[pl.ds(0, 8)], sr).wait()
        pltpu.make_async_copy(gbs[b], acc.at[dst_v.at[c]], sss[b]).wait()
        if c + 2 < WIN:
          fire_gather(c + 2, b)
        else:
          @pl.when(jnp.logical_not(last))
          def _():
            fire_gather(c + 2 - WIN, b)
      return carry

    lax.fori_loop(0, GROUPS, group, 0)

    plsc.subcore_barrier()

    pltpu.sync_copy(acc.at[pl.ds(base, ROWS_PER_SUB)],
                    out_hbm.at[cid, pl.ds(base, ROWS_PER_SUB)])
    pltpu.sync_copy(hist, cnt_hbm.at[wid])

  return k(x, ei3)


def _tc_combine_body(p_ref, c_ref, x_ref, wl_ref, wr_ref, b_ref, o_ref):
  s = p_ref[0] + p_ref[1]
  cnt = lax.dot_general(c_ref[...], jnp.ones((NW, 1), jnp.float32),
                        (((0,), (0,)), ((), ())),
                        preferred_element_type=jnp.float32)
  mean = s / jnp.maximum(cnt, 1.0)
  dims = (((1,), (1,)), ((), ()))
  o_ref[...] = (
      lax.dot_general(mean, wl_ref[...], dims,
                      preferred_element_type=jnp.float32)
      + b_ref[...]
      + lax.dot_general(x_ref[...], wr_ref[...], dims,
                        preferred_element_type=jnp.float32))


def _tc_combine(part, cnt, x, w_l, w_r, b_l):
  blk = 2048
  grid = pl.cdiv(N, blk)
  return pl.pallas_call(
      _tc_combine_body,
      grid=(grid,),
      in_specs=[
          pl.BlockSpec((NUM_CORES, blk, IN_CH), lambda i: (0, i, 0)),
          pl.BlockSpec((NW, blk), lambda i: (0, i)),
          pl.BlockSpec((blk, IN_CH), lambda i: (i, 0)),
          pl.BlockSpec((OUT_CH, IN_CH), lambda i: (0, 0)),
          pl.BlockSpec((OUT_CH, IN_CH), lambda i: (0, 0)),
          pl.BlockSpec((1, OUT_CH), lambda i: (0, 0)),
      ],
      out_specs=pl.BlockSpec((blk, OUT_CH), lambda i: (i, 0)),
      out_shape=jax.ShapeDtypeStruct((N, OUT_CH), jnp.float32),
  )(part, cnt, x, w_l, w_r, b_l)


def kernel(x, edge_index, W_l, b_l, W_r):
  ei3 = edge_index.reshape(2, NW * CPW, CHUNK)
  part, cnt = _sc_segment_sum(x, ei3)
  return _tc_combine(part, cnt, x, W_l, W_r, b_l.reshape(1, OUT_CH))


# speedup vs baseline: 16.3249x; 1.0391x over previous
import functools

import jax
import jax.numpy as jnp
from jax import lax
from jax.experimental import pallas as pl
from jax.experimental.pallas import tpu as pltpu
from jax.experimental.pallas import tpu_sc as plsc

N = 10000
E = 320000
IN_CH = 128
OUT_CH = 128

NPAD = 10112
ROWS_PER_SUB = 632
NUM_CORES = 2
NUM_SUBCORES = 16
NW = NUM_CORES * NUM_SUBCORES
CHUNK = 125
CPW = 80
WIN = 16
GROUPS = CPW // WIN


def _sc_segment_sum(x, ei3):
  mesh = plsc.VectorSubcoreMesh(
      core_axis_name="c", subcore_axis_name="s",
      num_cores=NUM_CORES, num_subcores=NUM_SUBCORES)

  @functools.partial(
      pl.kernel,
      out_type=(
          jax.ShapeDtypeStruct((NUM_CORES, NPAD, IN_CH), jnp.float32),
          jax.ShapeDtypeStruct((NW, NPAD), jnp.float32),
      ),
      mesh=mesh,
      compiler_params=pltpu.CompilerParams(use_tc_tiling_on_sc=False,
                                           needs_layout_passes=False),
      scratch_types=[
          pltpu.VMEM((WIN, CHUNK), jnp.int32),
          pltpu.VMEM((WIN, CHUNK), jnp.int32),
          pltpu.VMEM((CHUNK, IN_CH), jnp.float32),
          pltpu.VMEM((CHUNK, IN_CH), jnp.float32),
          pltpu.VMEM((NPAD,), jnp.float32),
          pltpu.VMEM_SHARED((NPAD, IN_CH), jnp.float32),
          pltpu.SemaphoreType.DMA,
          pltpu.SemaphoreType.DMA,
          pltpu.SemaphoreType.DMA,
          pltpu.SemaphoreType.DMA,
          pltpu.SemaphoreType.DMA,
      ],
  )
  def k(x_hbm, ei_hbm, out_hbm, cnt_hbm, src_v, dst_v,
        gb0, gb1, hist, acc, sg0, sg1, ss0, ss1, sr):
    src_hbm = ei_hbm.at[0]
    dst_hbm = ei_hbm.at[1]
    cid = lax.axis_index("c")
    sid = lax.axis_index("s")
    wid = cid * NUM_SUBCORES + sid
    base = sid * ROWS_PER_SUB

    zeros16 = jnp.zeros((16,), jnp.float32)
    ones16 = jnp.ones((16,), jnp.float32)
    tail_mask = lax.iota(jnp.int32, 16) >= 3

    gbs = (gb0, gb1)
    sgs = (sg0, sg1)
    sss = (ss0, ss1)

    def zrow(r, carry):
      for j in range(IN_CH // 16):
        gb0[r, pl.ds(j * 16, 16)] = zeros16
      return carry

    lax.fori_loop(0, 120, zrow, 0)

    for t in range(5):
      pltpu.async_copy(gb0.at[pl.ds(0, 120)],
                       acc.at[pl.ds(base + t * 120, 120)], sg0)
    pltpu.async_copy(gb0.at[pl.ds(0, 32)], acc.at[pl.ds(base + 600, 32)],
                     sg1)
    pltpu.async_copy(src_hbm.at[pl.ds(wid * CPW, WIN)], src_v, sr)
    pltpu.async_copy(dst_hbm.at[pl.ds(wid * CPW, WIN)], dst_v, sr)

    def hrow(r, carry):
      hist[pl.ds(pl.multiple_of(r * 16, 16), 16)] = zeros16
      return carry

    lax.fori_loop(0, NPAD // 16, hrow, 0)

    for t in range(5):
      pltpu.make_async_copy(gb0.at[pl.ds(0, 120)],
                            acc.at[pl.ds(base + t * 120, 120)], sg0).wait()
    pltpu.make_async_copy(gb0.at[pl.ds(0, 32)],
                          acc.at[pl.ds(base + 600, 32)], sg1).wait()
    pltpu.make_async_copy(src_hbm.at[pl.ds(wid * CPW, WIN)], src_v,
                          sr).wait()
    pltpu.make_async_copy(dst_hbm.at[pl.ds(wid * CPW, WIN)], dst_v,
                          sr).wait()

    plsc.subcore_barrier()

    def fire_gather(c, b):
      pltpu.async_copy(x_hbm.at[src_v.at[c, pl.ds(0, 64)]],
                       gbs[b].at[pl.ds(0, 64)], sgs[b])
      pltpu.async_copy(x_hbm.at[src_v.at[c, pl.ds(64, 61)]],
                       gbs[b].at[pl.ds(64, 61)], sgs[b])

    def wait_gather(c, b):
      pltpu.make_async_copy(x_hbm.at[src_v.at[c, pl.ds(0, 64)]],
                            gbs[b].at[pl.ds(0, 64)], sgs[b]).wait()
      pltpu.make_async_copy(x_hbm.at[src_v.at[c, pl.ds(64, 61)]],
                            gbs[b].at[pl.ds(64, 61)], sgs[b]).wait()

    fire_gather(0, 0)
    fire_gather(1, 1)

    def hist_chunk(c):
      for j in range(7):
        idx = dst_v[c, pl.ds(j * 16, 16)]
        plsc.addupdate_scatter(hist, [idx], ones16)
      idx = dst_v[c, pl.ds(109, 16)]
      plsc.addupdate_scatter(hist, [idx], ones16, mask=tail_mask)

    def group(i, carry):
      last = i == GROUPS - 1
      for c in range(WIN):
        b = c % 2
        wait_gather(c, b)
        pltpu.async_copy(gbs[b], acc.at[dst_v.at[c]], sss[b], add=True)
        if c == 0:
          row0 = wid * CPW + i * WIN + 8
          pltpu.async_copy(src_hbm.at[pl.ds(row0, 8)],
                           src_v.at[pl.ds(8, 8)], sr)
          pltpu.async_copy(dst_hbm.at[pl.ds(row0, 8)],
                           dst_v.at[pl.ds(8, 8)], sr)
        if c == 8:
          @pl.when(jnp.logical_not(last))
          def _():
            row0 = wid * CPW + (i + 1) * WIN
            pltpu.async_copy(src_hbm.at[pl.ds(row0, 8)],
                             src_v.at[pl.ds(0, 8)], sr)
            pltpu.async_copy(dst_hbm.at[pl.ds(row0, 8)],
                             dst_v.at[pl.ds(0, 8)], sr)
        hist_chunk(c)
        if c == 5:
          pltpu.make_async_copy(src_hbm.at[pl.ds(0, 8)],
                                src_v.at[pl.ds(8, 8)], sr).wait()
          pltpu.make_async_copy(dst_hbm.at[pl.ds(0, 8)],
                                dst_v.at[pl.ds(8, 8)], sr).wait()
        if c == 13:
          @pl.when(jnp.logical_not(last))
          def _():
            pltpu.make_async_copy(src_hbm.at[pl.ds(0, 8)],
                                  src_v.at[pl.ds(0, 8)], sr).wait()
            pltpu.make_async_copy(dst_hbm.at[pl.ds(0, 8)],
                                  dst_v.at[pl.ds(0, 8)], sr).wait()
        pltpu.make_async_copy(gbs[b], acc.at[dst_v.at[c]], sss[b]).wait()
        if c + 2 < WIN:
          fire_gather(c + 2, b)
        else:
          @pl.when(jnp.logical_not(last))
          def _():
            fire_gather(c + 2 - WIN, b)
      return carry

    lax.fori_loop(0, GROUPS, group, 0)

    plsc.subcore_barrier()

    pltpu.sync_copy(acc.at[pl.ds(base, ROWS_PER_SUB)],
                    out_hbm.at[cid, pl.ds(base, ROWS_PER_SUB)])
    pltpu.sync_copy(hist, cnt_hbm.at[wid])

  return k(x, ei3)


def _tc_combine_body(p_ref, c_ref, x_ref, wl_ref, wr_ref, b_ref, o_ref):
  s = p_ref[0] + p_ref[1]
  cnt = lax.dot_general(c_ref[...], jnp.ones((NW, 1), jnp.float32),
                        (((0,), (0,)), ((), ())),
                        preferred_element_type=jnp.float32)
  mean = s / jnp.maximum(cnt, 1.0)
  dims = (((1,), (1,)), ((), ()))
  o_ref[...] = (
      lax.dot_general(mean, wl_ref[...], dims,
                      preferred_element_type=jnp.float32)
      + b_ref[...]
      + lax.dot_general(x_ref[...], wr_ref[...], dims,
                        preferred_element_type=jnp.float32))


def _tc_combine(part, cnt, x, w_l, w_r, b_l):
  blk = 2560
  grid = pl.cdiv(N, blk)
  return pl.pallas_call(
      _tc_combine_body,
      grid=(grid,),
      in_specs=[
          pl.BlockSpec((NUM_CORES, blk, IN_CH), lambda i: (0, i, 0)),
          pl.BlockSpec((NW, blk), lambda i: (0, i)),
          pl.BlockSpec((blk, IN_CH), lambda i: (i, 0)),
          pl.BlockSpec((OUT_CH, IN_CH), lambda i: (0, 0)),
          pl.BlockSpec((OUT_CH, IN_CH), lambda i: (0, 0)),
          pl.BlockSpec((1, OUT_CH), lambda i: (0, 0)),
      ],
      out_specs=pl.BlockSpec((blk, OUT_CH), lambda i: (i, 0)),
      out_shape=jax.ShapeDtypeStruct((N, OUT_CH), jnp.float32),
  )(part, cnt, x, w_l, w_r, b_l)


def kernel(x, edge_index, W_l, b_l, W_r):
  ei3 = edge_index.reshape(2, NW * CPW, CHUNK)
  part, cnt = _sc_segment_sum(x, ei3)
  return _tc_combine(part, cnt, x, W_l, W_r, b_l.reshape(1, OUT_CH))


# speedup vs baseline: 16.4244x; 1.0061x over previous
import functools

import jax
import jax.numpy as jnp
from jax import lax
from jax.experimental import pallas as pl
from jax.experimental.pallas import tpu as pltpu
from jax.experimental.pallas import tpu_sc as plsc

N = 10000
E = 320000
IN_CH = 128
OUT_CH = 128

NPAD = 10112
ROWS_PER_SUB = 632
NUM_CORES = 2
NUM_SUBCORES = 16
NW = NUM_CORES * NUM_SUBCORES
CHUNK = 125
CPW = 80
WIN = 16
GROUPS = CPW // WIN


def _sc_segment_sum(x, ei3):
  mesh = plsc.VectorSubcoreMesh(
      core_axis_name="c", subcore_axis_name="s",
      num_cores=NUM_CORES, num_subcores=NUM_SUBCORES)

  @functools.partial(
      pl.kernel,
      out_type=(
          jax.ShapeDtypeStruct((NUM_CORES, NPAD, IN_CH), jnp.float32),
          jax.ShapeDtypeStruct((NW, NPAD), jnp.float32),
      ),
      mesh=mesh,
      compiler_params=pltpu.CompilerParams(use_tc_tiling_on_sc=False,
                                           needs_layout_passes=False),
      scratch_types=[
          pltpu.VMEM((WIN, CHUNK), jnp.int32),
          pltpu.VMEM((WIN, CHUNK), jnp.int32),
          pltpu.VMEM((CHUNK, IN_CH), jnp.float32),
          pltpu.VMEM((CHUNK, IN_CH), jnp.float32),
          pltpu.VMEM((NPAD,), jnp.float32),
          pltpu.VMEM_SHARED((NPAD, IN_CH), jnp.float32),
          pltpu.SemaphoreType.DMA,
          pltpu.SemaphoreType.DMA,
          pltpu.SemaphoreType.DMA,
          pltpu.SemaphoreType.DMA,
          pltpu.SemaphoreType.DMA,
      ],
  )
  def k(x_hbm, ei_hbm, out_hbm, cnt_hbm, src_v, dst_v,
        gb0, gb1, hist, acc, sg0, sg1, ss0, ss1, sr):
    src_hbm = ei_hbm.at[0]
    dst_hbm = ei_hbm.at[1]
    cid = lax.axis_index("c")
    sid = lax.axis_index("s")
    wid = cid * NUM_SUBCORES + sid
    base = sid * ROWS_PER_SUB

    zeros16 = jnp.zeros((16,), jnp.float32)
    ones16 = jnp.ones((16,), jnp.float32)
    tail_mask = lax.iota(jnp.int32, 16) >= 3

    gbs = (gb0, gb1)
    sgs = (sg0, sg1)
    sss = (ss0, ss1)

    def zrow(r, carry):
      for j in range(IN_CH // 16):
        gb0[r, pl.ds(j * 16, 16)] = zeros16
      return carry

    lax.fori_loop(0, 120, zrow, 0)

    for t in range(5):
      pltpu.async_copy(gb0.at[pl.ds(0, 120)],
                       acc.at[pl.ds(base + t * 120, 120)], sg0)
    pltpu.async_copy(gb0.at[pl.ds(0, 32)], acc.at[pl.ds(base + 600, 32)],
                     sg1)
    pltpu.async_copy(src_hbm.at[pl.ds(wid * CPW, WIN)], src_v, sr)
    pltpu.async_copy(dst_hbm.at[pl.ds(wid * CPW, WIN)], dst_v, sr)

    def hrow(r, carry):
      hist[pl.ds(pl.multiple_of(r * 16, 16), 16)] = zeros16
      return carry

    lax.fori_loop(0, NPAD // 16, hrow, 0)

    for t in range(5):
      pltpu.make_async_copy(gb0.at[pl.ds(0, 120)],
                            acc.at[pl.ds(base + t * 120, 120)], sg0).wait()
    pltpu.make_async_copy(gb0.at[pl.ds(0, 32)],
                          acc.at[pl.ds(base + 600, 32)], sg1).wait()
    pltpu.make_async_copy(src_hbm.at[pl.ds(wid * CPW, WIN)], src_v,
                          sr).wait()
    pltpu.make_async_copy(dst_hbm.at[pl.ds(wid * CPW, WIN)], dst_v,
                          sr).wait()

    plsc.subcore_barrier()

    def fire_gather(c, b):
      pltpu.async_copy(x_hbm.at[src_v.at[c, pl.ds(0, 64)]],
                       gbs[b].at[pl.ds(0, 64)], sgs[b])
      pltpu.async_copy(x_hbm.at[src_v.at[c, pl.ds(64, 61)]],
                       gbs[b].at[pl.ds(64, 61)], sgs[b])

    def wait_gather(c, b):
      pltpu.make_async_copy(x_hbm.at[src_v.at[c, pl.ds(0, 64)]],
                            gbs[b].at[pl.ds(0, 64)], sgs[b]).wait()
      pltpu.make_async_copy(x_hbm.at[src_v.at[c, pl.ds(64, 61)]],
                            gbs[b].at[pl.ds(64, 61)], sgs[b]).wait()

    fire_gather(0, 0)
    fire_gather(1, 1)

    def hist_chunk(c):
      for j in range(7):
        idx = dst_v[c, pl.ds(j * 16, 16)]
        plsc.addupdate_scatter(hist, [idx], ones16)
      idx = dst_v[c, pl.ds(109, 16)]
      plsc.addupdate_scatter(hist, [idx], ones16, mask=tail_mask)

    def group(i, carry):
      last = i == GROUPS - 1
      for c in range(WIN):
        b = c % 2
        wait_gather(c, b)
        pltpu.async_copy(gbs[b], acc.at[dst_v.at[c]], sss[b], add=True)
        if c == 0:
          row0 = wid * CPW + i * WIN + 8
          pltpu.async_copy(src_hbm.at[pl.ds(row0, 8)],
                           src_v.at[pl.ds(8, 8)], sr)
          pltpu.async_copy(dst_hbm.at[pl.ds(row0, 8)],
                           dst_v.at[pl.ds(8, 8)], sr)
        if c == 8:
          @pl.when(jnp.logical_not(last))
          def _():
            row0 = wid * CPW + (i + 1) * WIN
            pltpu.async_copy(src_hbm.at[pl.ds(row0, 8)],
                             src_v.at[pl.ds(0, 8)], sr)
            pltpu.async_copy(dst_hbm.at[pl.ds(row0, 8)],
                             dst_v.at[pl.ds(0, 8)], sr)
        hist_chunk(c)
        if c == 5:
          pltpu.make_async_copy(src_hbm.at[pl.ds(0, 8)],
                                src_v.at[pl.ds(8, 8)], sr).wait()
          pltpu.make_async_copy(dst_hbm.at[pl.ds(0, 8)],
                                dst_v.at[pl.ds(8, 8)], sr).wait()
        if c == 13:
          @pl.when(jnp.logical_not(last))
          def _():
            pltpu.make_async_copy(src_hbm.at[pl.ds(0, 8)],
                                  src_v.at[pl.ds(0, 8)], sr).wait()
            pltpu.make_async_copy(dst_hbm.at[pl.ds(0, 8)],
                                  dst_v.at[pl.ds(0, 8)], sr).wait()
        pltpu.make_async_copy(gbs[b], acc.at[dst_v.at[c]], sss[b]).wait()
        if c + 2 < WIN:
          fire_gather(c + 2, b)
        else:
          @pl.when(jnp.logical_not(last))
          def _():
            fire_gather(c + 2 - WIN, b)
      return carry

    lax.fori_loop(0, GROUPS, group, 0)

    plsc.subcore_barrier()

    pltpu.async_copy(acc.at[pl.ds(base, ROWS_PER_SUB)],
                     out_hbm.at[cid, pl.ds(base, ROWS_PER_SUB)], sg0)
    pltpu.async_copy(hist, cnt_hbm.at[wid], sg1)
    pltpu.make_async_copy(acc.at[pl.ds(base, ROWS_PER_SUB)],
                          out_hbm.at[cid, pl.ds(base, ROWS_PER_SUB)],
                          sg0).wait()
    pltpu.make_async_copy(hist, cnt_hbm.at[wid], sg1).wait()

  return k(x, ei3)


def _tc_combine_body(p_ref, c_ref, x_ref, wl_ref, wr_ref, b_ref, o_ref):
  s = p_ref[0] + p_ref[1]
  cnt = lax.dot_general(c_ref[...], jnp.ones((NW, 1), jnp.float32),
                        (((0,), (0,)), ((), ())),
                        preferred_element_type=jnp.float32)
  mean = s / jnp.maximum(cnt, 1.0)
  dims = (((1,), (1,)), ((), ()))
  o_ref[...] = (
      lax.dot_general(mean, wl_ref[...], dims,
                      preferred_element_type=jnp.float32)
      + b_ref[...]
      + lax.dot_general(x_ref[...], wr_ref[...], dims,
                        preferred_element_type=jnp.float32))


def _tc_combine(part, cnt, x, w_l, w_r, b_l):
  blk = 2560
  grid = pl.cdiv(N, blk)
  return pl.pallas_call(
      _tc_combine_body,
      grid=(grid,),
      in_specs=[
          pl.BlockSpec((NUM_CORES, blk, IN_CH), lambda i: (0, i, 0)),
          pl.BlockSpec((NW, blk), lambda i: (0, i)),
          pl.BlockSpec((blk, IN_CH), lambda i: (i, 0)),
          pl.BlockSpec((OUT_CH, IN_CH), lambda i: (0, 0)),
          pl.BlockSpec((OUT_CH, IN_CH), lambda i: (0, 0)),
          pl.BlockSpec((1, OUT_CH), lambda i: (0, 0)),
      ],
      out_specs=pl.BlockSpec((blk, OUT_CH), lambda i: (i, 0)),
      out_shape=jax.ShapeDtypeStruct((N, OUT_CH), jnp.float32),
  )(part, cnt, x, w_l, w_r, b_l)


def kernel(x, edge_index, W_l, b_l, W_r):
  ei3 = edge_index.reshape(2, NW * CPW, CHUNK)
  part, cnt = _sc_segment_sum(x, ei3)
  return _tc_combine(part, cnt, x, W_l, W_r, b_l.reshape(1, OUT_CH))
